# baseline (speedup 1.0000x reference)
import functools

import jax
import jax.numpy as jnp
from jax.experimental import pallas as pl
from jax.experimental.pallas import tpu as pltpu

_INTERPRET = False

_LOG2E = 1.4426950408889634
_CONTRACT_LAST = (((1,), (1,)), ((), ()))
_VMEM_LIM = 110 * 1024 * 1024


def _bf(x):
    return x.astype(jnp.bfloat16)


def _silu(v):
    return v * jax.nn.sigmoid(v)


def _rms_bf16(xv, nw):
    ms = jnp.mean(xv * xv, axis=-1, keepdims=True)
    return _bf(xv * jax.lax.rsqrt(ms + 1e-5) * nw)


def _gather_body(ids_ref, emb_ref, out_ref, sem):
    n = out_ref.shape[0]

    def issue(i, _):
        idx = ids_ref[i]
        pltpu.make_async_copy(emb_ref.at[pl.ds(idx, 1), :],
                              out_ref.at[pl.ds(i, 1), :], sem).start()
        return 0

    jax.lax.fori_loop(0, n, issue, 0)

    def drain(i, _):
        pltpu.make_async_copy(emb_ref.at[pl.ds(0, 1), :],
                              out_ref.at[pl.ds(0, 1), :], sem).wait()
        return 0

    jax.lax.fori_loop(0, n, drain, 0)


def _embed_gather(ids_flat, embed):
    m = ids_flat.shape[0]
    dm = embed.shape[1]
    return pl.pallas_call(
        _gather_body,
        out_shape=jax.ShapeDtypeStruct((m, dm), jnp.float32),
        in_specs=[pl.BlockSpec(memory_space=pltpu.SMEM),
                  pl.BlockSpec(memory_space=pl.ANY)],
        out_specs=pl.BlockSpec(memory_space=pltpu.VMEM),
        scratch_shapes=[pltpu.SemaphoreType.DMA],
        name="embed_gather",
        interpret=_INTERPRET,
    )(ids_flat, embed)


def _layers_body(x0_ref, nw_ref, win_ref, cw_ref, cb_ref, wdtr_ref, wb_ref,
                 wc_ref, wdt_ref, dtb_ref, alog_ref, d_ref, wo_ref, wf_ref,
                 hf_ref,
                 x_ref, u_ref, zs_ref, dts_ref, yg_ref,
                 bmt_ref, cmt_ref, bm8_ref, cm8_ref,
                 *, seg, nl):
    l = pl.program_id(0)
    mb = u_ref.shape[0]
    di = u_ref.shape[2]
    m = mb * 8
    ds = alog_ref.shape[1]
    nblk = seg // 8
    nbatch = m // seg

    @pl.when(l == 0)
    def _():
        x_ref[...] = x0_ref[...]

    hb = _rms_bf16(x_ref[...], nw_ref[0])
    xz_u = jax.lax.dot_general(hb, win_ref[0, 0:di], _CONTRACT_LAST,
                               preferred_element_type=jnp.float32)
    xz_z = jax.lax.dot_general(hb, win_ref[0, di:2 * di], _CONTRACT_LAST,
                               preferred_element_type=jnp.float32)
    dc = cw_ref.shape[1]
    row = jax.lax.broadcasted_iota(jnp.int32, (m, 1), 0)
    pos = jax.lax.rem(row, seg)
    uc = xz_u * cw_ref[0, dc - 1:dc, :]
    for s in range(1, dc):
        shifted = jnp.concatenate(
            [jnp.zeros((s, di), jnp.float32), xz_u[:-s, :]], axis=0)
        shifted = jnp.where(pos >= s, shifted, 0.0)
        uc = uc + shifted * cw_ref[0, dc - 1 - s:dc - s, :]
    u = _silu(uc + cb_ref[0])
    u_ref[...] = u.reshape(mb, 8, di)
    zs_ref[...] = _bf(_silu(xz_z)).reshape(mb, 8, di)

    ub = _bf(u)
    dtr = jax.lax.dot_general(ub, wdtr_ref[0], _CONTRACT_LAST,
                              preferred_element_type=jnp.float32)
    bmt_ref[...] = jax.lax.dot_general(wb_ref[0], ub, _CONTRACT_LAST,
                                       preferred_element_type=jnp.float32)
    cmt_ref[...] = jax.lax.dot_general(wc_ref[0], ub, _CONTRACT_LAST,
                                       preferred_element_type=jnp.float32)
    dtx = jax.lax.dot_general(_bf(dtr), wdt_ref[0], _CONTRACT_LAST,
                              preferred_element_type=jnp.float32)
    dtx = dtx + dtb_ref[0]
    dt = jnp.where(dtx > 20.0, dtx, jnp.log1p(jnp.exp(dtx)))
    dts_ref[...] = dt.reshape(mb, 8, di)
    for i in range(mb):
        bm8_ref[i] = bmt_ref[:, 8 * i:8 * i + 8]
        cm8_ref[i] = cmt_ref[:, 8 * i:8 * i + 8]

    a_sc = (-_LOG2E) * jnp.exp(alog_ref[0])
    dvec = d_ref[0]

    nrep = di // 128

    def batch_block(base, h):
        dt8 = dts_ref[base]
        u8 = u_ref[base]
        bc8 = bm8_ref[base]
        cc8 = cm8_ref[base]
        ys = []
        for j in range(8):
            dt_row = dt8[j:j + 1, :]
            dtb = pltpu.repeat(jnp.broadcast_to(dt_row, (8, di)), 2, axis=0)
            dtu = dt_row * u8[j:j + 1, :]
            dtub = pltpu.repeat(jnp.broadcast_to(dtu, (8, di)), 2, axis=0)
            bcf = pltpu.repeat(
                jnp.broadcast_to(bc8[:, j:j + 1], (ds, 128)), nrep, axis=1)
            ccf = pltpu.repeat(
                jnp.broadcast_to(cc8[:, j:j + 1], (ds, 128)), nrep, axis=1)
            a = jnp.exp2(a_sc * dtb)
            h = a * h + bcf * dtub
            ys.append(jnp.sum(ccf * h, axis=0, keepdims=True))
        y8 = jnp.concatenate(ys, axis=0)
        yg_ref[base] = _bf((y8 + u8 * dvec) *
                           zs_ref[base].astype(jnp.float32))
        return h

    def body(blk, carry):
        return tuple(
            batch_block(b * nblk + blk, carry[b]) for b in range(nbatch))

    z = jnp.zeros((ds, di), jnp.float32)
    jax.lax.fori_loop(0, nblk, body, (z,) * nbatch)

    xn = x_ref[...] + jax.lax.dot_general(
        yg_ref[...].reshape(m, di), wo_ref[0], _CONTRACT_LAST,
        preferred_element_type=jnp.float32)
    x_ref[...] = xn

    @pl.when(l == nl - 1)
    def _():
        hf_ref[...] = _rms_bf16(xn, wf_ref[0])


def _layers(x0, norm_w, win_bf, cw, cb, wdtr_bf, wb_bf, wc_bf, wdt_bf, dtb,
            alogT, dmat, wo_bf, norm_f_w, *, seg):
    m, dm = x0.shape
    nl, ds, di = alogT.shape
    dtrk = wdt_bf.shape[2]
    dc = cw.shape[1]
    mb = m // 8
    kern = functools.partial(_layers_body, seg=seg, nl=nl)
    return pl.pallas_call(
        kern,
        grid=(nl,),
        in_specs=[
            pl.BlockSpec((m, dm), lambda l: (0, 0)),
            pl.BlockSpec((1, 1, dm), lambda l: (l, 0, 0)),
            pl.BlockSpec((1, 2 * di, dm), lambda l: (l, 0, 0)),
            pl.BlockSpec((1, dc, di), lambda l: (l, 0, 0)),
            pl.BlockSpec((1, 1, di), lambda l: (l, 0, 0)),
            pl.BlockSpec((1, dtrk, di), lambda l: (l, 0, 0)),
            pl.BlockSpec((1, ds, di), lambda l: (l, 0, 0)),
            pl.BlockSpec((1, ds, di), lambda l: (l, 0, 0)),
            pl.BlockSpec((1, di, dtrk), lambda l: (l, 0, 0)),
            pl.BlockSpec((1, 1, di), lambda l: (l, 0, 0)),
            pl.BlockSpec((1, ds, di), lambda l: (l, 0, 0)),
            pl.BlockSpec((1, 1, di), lambda l: (l, 0, 0)),
            pl.BlockSpec((1, dm, di), lambda l: (l, 0, 0)),
            pl.BlockSpec((1, 1, dm), lambda l: (0, 0, 0)),
        ],
        out_specs=pl.BlockSpec((m, dm), lambda l: (0, 0)),
        out_shape=jax.ShapeDtypeStruct((m, dm), jnp.bfloat16),
        scratch_shapes=[
            pltpu.VMEM((m, dm), jnp.float32),
            pltpu.VMEM((mb, 8, di), jnp.float32),
            pltpu.VMEM((mb, 8, di), jnp.bfloat16),
            pltpu.VMEM((mb, 8, di), jnp.float32),
            pltpu.VMEM((mb, 8, di), jnp.bfloat16),
            pltpu.VMEM((ds, m), jnp.float32),
            pltpu.VMEM((ds, m), jnp.float32),
            pltpu.VMEM((mb, ds, 8), jnp.float32),
            pltpu.VMEM((mb, ds, 8), jnp.float32),
        ],
        compiler_params=pltpu.CompilerParams(
            dimension_semantics=("arbitrary",),
            vmem_limit_bytes=_VMEM_LIM,
        ),
        name="mamba_layers",
        interpret=_INTERPRET,
    )(x0, norm_w, win_bf, cw, cb, wdtr_bf, wb_bf, wc_bf, wdt_bf, dtb,
      alogT, dmat, wo_bf, norm_f_w.reshape(1, 1, dm))


def _lmhead_body(h_ref, e_ref, o_ref):
    o_ref[...] = jax.lax.dot_general(
        h_ref[...], _bf(e_ref[...]), _CONTRACT_LAST,
        preferred_element_type=jnp.float32)


def _lmhead(hf, embed, *, vtile):
    m, dm = hf.shape
    v = embed.shape[0]
    nv = v // vtile
    return pl.pallas_call(
        _lmhead_body,
        grid=(nv,),
        in_specs=[
            pl.BlockSpec((m, dm), lambda i: (0, 0)),
            pl.BlockSpec((vtile, dm), lambda i: (i, 0)),
        ],
        out_specs=pl.BlockSpec((m, vtile), lambda i: (0, i)),
        out_shape=jax.ShapeDtypeStruct((m, v), jnp.float32),
        compiler_params=pltpu.CompilerParams(
            dimension_semantics=("arbitrary",),
            vmem_limit_bytes=_VMEM_LIM,
        ),
        name="lm_head",
        interpret=_INTERPRET,
    )(hf, embed)


def kernel(input_ids, embed, norm_w, in_proj_w, conv_w, conv_b, x_proj_w,
           dt_proj_w, dt_proj_b, A_log, D, out_proj_w, norm_f_w):
    bsz, seg = input_ids.shape
    v, dm = embed.shape
    nl, di, ds = A_log.shape
    dtr = dt_proj_w.shape[2]
    m = bsz * seg

    cw = jnp.swapaxes(conv_w[:, :, 0, :], 1, 2)
    alogT = jnp.swapaxes(A_log, 1, 2)
    wdtr = _bf(x_proj_w[:, :dtr, :])
    wb = _bf(x_proj_w[:, dtr:dtr + ds, :])
    wc = _bf(x_proj_w[:, dtr + ds:, :])
    win_bf = _bf(in_proj_w)
    wo_bf = _bf(out_proj_w)
    wdt_bf = _bf(dt_proj_w)

    x0 = _embed_gather(input_ids.reshape(m), embed)
    hf = _layers(x0, norm_w.reshape(nl, 1, dm), win_bf, cw,
                 conv_b.reshape(nl, 1, di), wdtr, wb, wc, wdt_bf,
                 dt_proj_b.reshape(nl, 1, di), alogT, D.reshape(nl, 1, di),
                 wo_bf, norm_f_w, seg=seg)
    logits = _lmhead(hf, embed, vtile=1280)
    return logits.reshape(bsz, seg, v)


# speedup vs baseline: 1.0170x; 1.0170x over previous
import functools

import jax
import jax.numpy as jnp
from jax.experimental import pallas as pl
from jax.experimental.pallas import tpu as pltpu

_INTERPRET = False

_LOG2E = 1.4426950408889634
_CONTRACT_LAST = (((1,), (1,)), ((), ()))
_VMEM_LIM = 110 * 1024 * 1024


def _bf(x):
    return x.astype(jnp.bfloat16)


def _silu(v):
    return v * jax.nn.sigmoid(v)


def _rms_bf16(xv, nw):
    ms = jnp.mean(xv * xv, axis=-1, keepdims=True)
    return _bf(xv * jax.lax.rsqrt(ms + 1e-5) * nw)


def _gather_body(ids_ref, emb_ref, out_ref, sem):
    n = out_ref.shape[0]

    def issue(i, _):
        idx = ids_ref[i]
        pltpu.make_async_copy(emb_ref.at[pl.ds(idx, 1), :],
                              out_ref.at[pl.ds(i, 1), :], sem).start()
        return 0

    jax.lax.fori_loop(0, n, issue, 0)

    def drain(i, _):
        pltpu.make_async_copy(emb_ref.at[pl.ds(0, 1), :],
                              out_ref.at[pl.ds(0, 1), :], sem).wait()
        return 0

    jax.lax.fori_loop(0, n, drain, 0)


def _embed_gather(ids_flat, embed):
    m = ids_flat.shape[0]
    dm = embed.shape[1]
    return pl.pallas_call(
        _gather_body,
        out_shape=jax.ShapeDtypeStruct((m, dm), jnp.float32),
        in_specs=[pl.BlockSpec(memory_space=pltpu.SMEM),
                  pl.BlockSpec(memory_space=pl.ANY)],
        out_specs=pl.BlockSpec(memory_space=pltpu.VMEM),
        scratch_shapes=[pltpu.SemaphoreType.DMA],
        name="embed_gather",
        interpret=_INTERPRET,
    )(ids_flat, embed)


def _layers_body(x0_ref, nw_ref, win_ref, cw_ref, cb_ref, wdtr_ref, wb_ref,
                 wc_ref, wdt_ref, dtb_ref, alog_ref, d_ref, wo_ref, wf_ref,
                 hf_ref,
                 x_ref, u_ref, zs_ref, dts_ref, yg_ref,
                 bmt_ref, cmt_ref, bm8_ref, cm8_ref,
                 *, seg, nl):
    l = pl.program_id(0)
    mb = u_ref.shape[0]
    di = u_ref.shape[2]
    m = mb * 8
    ds = alog_ref.shape[1]
    nblk = seg // 8
    nbatch = m // seg

    @pl.when(l == 0)
    def _():
        x_ref[...] = x0_ref[...]

    hb = _rms_bf16(x_ref[...], nw_ref[0])
    xz_u = jax.lax.dot_general(hb, win_ref[0, 0:di], _CONTRACT_LAST,
                               preferred_element_type=jnp.float32)
    xz_z = jax.lax.dot_general(hb, win_ref[0, di:2 * di], _CONTRACT_LAST,
                               preferred_element_type=jnp.float32)
    dc = cw_ref.shape[1]
    row = jax.lax.broadcasted_iota(jnp.int32, (m, 1), 0)
    pos = jax.lax.rem(row, seg)
    uc = xz_u * cw_ref[0, dc - 1:dc, :]
    for s in range(1, dc):
        shifted = jnp.concatenate(
            [jnp.zeros((s, di), jnp.float32), xz_u[:-s, :]], axis=0)
        shifted = jnp.where(pos >= s, shifted, 0.0)
        uc = uc + shifted * cw_ref[0, dc - 1 - s:dc - s, :]
    u = _silu(uc + cb_ref[0])
    u_ref[...] = u.reshape(mb, 8, di)
    zs_ref[...] = _bf(_silu(xz_z)).reshape(mb, 8, di)

    ub = _bf(u)
    dtr = jax.lax.dot_general(ub, wdtr_ref[0], _CONTRACT_LAST,
                              preferred_element_type=jnp.float32)
    bmt_ref[...] = jax.lax.dot_general(wb_ref[0], ub, _CONTRACT_LAST,
                                       preferred_element_type=jnp.float32)
    cmt_ref[...] = jax.lax.dot_general(wc_ref[0], ub, _CONTRACT_LAST,
                                       preferred_element_type=jnp.float32)
    dtx = jax.lax.dot_general(_bf(dtr), wdt_ref[0], _CONTRACT_LAST,
                              preferred_element_type=jnp.float32)
    dtx = dtx + dtb_ref[0]
    dt = jnp.where(dtx > 20.0, dtx, jnp.log1p(jnp.exp(dtx)))
    dts_ref[...] = dt.reshape(mb, 8, di)
    for i in range(mb):
        bm8_ref[i] = bmt_ref[:, 8 * i:8 * i + 8]
        cm8_ref[i] = cmt_ref[:, 8 * i:8 * i + 8]

    a_sc = (-_LOG2E) * jnp.exp(alog_ref[0])
    dvec = d_ref[0]

    nrep = di // 128

    def batch_block(base, h):
        dt8 = dts_ref[base]
        u8 = u_ref[base]
        bc8 = bm8_ref[base]
        cc8 = cm8_ref[base]
        ys = []
        for j in range(8):
            dt_row = dt8[j:j + 1, :]
            dtb = pltpu.repeat(jnp.broadcast_to(dt_row, (8, di)), 2, axis=0)
            dtu = dt_row * u8[j:j + 1, :]
            dtub = pltpu.repeat(jnp.broadcast_to(dtu, (8, di)), 2, axis=0)
            bcf = pltpu.repeat(
                jnp.broadcast_to(bc8[:, j:j + 1], (ds, 128)), nrep, axis=1)
            ccf = pltpu.repeat(
                jnp.broadcast_to(cc8[:, j:j + 1], (ds, 128)), nrep, axis=1)
            a = jnp.exp2(a_sc * dtb)
            h = a * h + bcf * dtub
            ys.append(jnp.sum(ccf * h, axis=0, keepdims=True))
        y8 = jnp.concatenate(ys, axis=0)
        yg_ref[base] = _bf((y8 + u8 * dvec) *
                           zs_ref[base].astype(jnp.float32))
        return h

    def body(blk, carry):
        return tuple(
            batch_block(b * nblk + blk, carry[b]) for b in range(nbatch))

    z = jnp.zeros((ds, di), jnp.float32)
    jax.lax.fori_loop(0, nblk, body, (z,) * nbatch)

    xn = x_ref[...] + jax.lax.dot_general(
        yg_ref[...].reshape(m, di), wo_ref[0], _CONTRACT_LAST,
        preferred_element_type=jnp.float32)
    x_ref[...] = xn

    @pl.when(l == nl - 1)
    def _():
        hf_ref[...] = _rms_bf16(xn, wf_ref[0])


def _layers(x0, norm_w, win_bf, cw, cb, wdtr_bf, wb_bf, wc_bf, wdt_bf, dtb,
            alogT, dmat, wo_bf, norm_f_w, *, seg):
    m, dm = x0.shape
    nl, ds, di = alogT.shape
    dtrk = wdt_bf.shape[2]
    dc = cw.shape[1]
    mb = m // 8
    kern = functools.partial(_layers_body, seg=seg, nl=nl)
    return pl.pallas_call(
        kern,
        grid=(nl,),
        in_specs=[
            pl.BlockSpec((m, dm), lambda l: (0, 0)),
            pl.BlockSpec((1, 1, dm), lambda l: (l, 0, 0)),
            pl.BlockSpec((1, 2 * di, dm), lambda l: (l, 0, 0)),
            pl.BlockSpec((1, dc, di), lambda l: (l, 0, 0)),
            pl.BlockSpec((1, 1, di), lambda l: (l, 0, 0)),
            pl.BlockSpec((1, dtrk, di), lambda l: (l, 0, 0)),
            pl.BlockSpec((1, ds, di), lambda l: (l, 0, 0)),
            pl.BlockSpec((1, ds, di), lambda l: (l, 0, 0)),
            pl.BlockSpec((1, di, dtrk), lambda l: (l, 0, 0)),
            pl.BlockSpec((1, 1, di), lambda l: (l, 0, 0)),
            pl.BlockSpec((1, ds, di), lambda l: (l, 0, 0)),
            pl.BlockSpec((1, 1, di), lambda l: (l, 0, 0)),
            pl.BlockSpec((1, dm, di), lambda l: (l, 0, 0)),
            pl.BlockSpec((1, 1, dm), lambda l: (0, 0, 0)),
        ],
        out_specs=pl.BlockSpec((m, dm), lambda l: (0, 0)),
        out_shape=jax.ShapeDtypeStruct((m, dm), jnp.bfloat16),
        scratch_shapes=[
            pltpu.VMEM((m, dm), jnp.float32),
            pltpu.VMEM((mb, 8, di), jnp.float32),
            pltpu.VMEM((mb, 8, di), jnp.bfloat16),
            pltpu.VMEM((mb, 8, di), jnp.float32),
            pltpu.VMEM((mb, 8, di), jnp.bfloat16),
            pltpu.VMEM((ds, m), jnp.float32),
            pltpu.VMEM((ds, m), jnp.float32),
            pltpu.VMEM((mb, ds, 8), jnp.float32),
            pltpu.VMEM((mb, ds, 8), jnp.float32),
        ],
        compiler_params=pltpu.CompilerParams(
            dimension_semantics=("arbitrary",),
            vmem_limit_bytes=_VMEM_LIM,
        ),
        name="mamba_layers",
        interpret=_INTERPRET,
    )(x0, norm_w, win_bf, cw, cb, wdtr_bf, wb_bf, wc_bf, wdt_bf, dtb,
      alogT, dmat, wo_bf, norm_f_w.reshape(1, 1, dm))


def _lmhead_body(h_ref, e_ref, o_ref):
    o_ref[...] = jax.lax.dot_general(
        h_ref[...], _bf(e_ref[...]), _CONTRACT_LAST,
        preferred_element_type=jnp.float32)


def _lmhead(hf, embed, *, vtile):
    m, dm = hf.shape
    v = embed.shape[0]
    nv = v // vtile
    return pl.pallas_call(
        _lmhead_body,
        grid=(nv,),
        in_specs=[
            pl.BlockSpec((m, dm), lambda i: (0, 0)),
            pl.BlockSpec((vtile, dm), lambda i: (i, 0)),
        ],
        out_specs=pl.BlockSpec((m, vtile), lambda i: (0, i)),
        out_shape=jax.ShapeDtypeStruct((m, v), jnp.float32),
        compiler_params=pltpu.CompilerParams(
            dimension_semantics=("arbitrary",),
            vmem_limit_bytes=_VMEM_LIM,
        ),
        name="lm_head",
        interpret=_INTERPRET,
    )(hf, embed)


def kernel(input_ids, embed, norm_w, in_proj_w, conv_w, conv_b, x_proj_w,
           dt_proj_w, dt_proj_b, A_log, D, out_proj_w, norm_f_w):
    bsz, seg = input_ids.shape
    v, dm = embed.shape
    nl, di, ds = A_log.shape
    dtr = dt_proj_w.shape[2]
    m = bsz * seg

    cw = jnp.swapaxes(conv_w[:, :, 0, :], 1, 2)
    alogT = jnp.swapaxes(A_log, 1, 2)
    wdtr = _bf(x_proj_w[:, :dtr, :])
    wb = _bf(x_proj_w[:, dtr:dtr + ds, :])
    wc = _bf(x_proj_w[:, dtr + ds:, :])
    win_bf = _bf(in_proj_w)
    wo_bf = _bf(out_proj_w)
    wdt_bf = _bf(dt_proj_w)

    x0 = _embed_gather(input_ids.reshape(m), embed)
    hf = _layers(x0, norm_w.reshape(nl, 1, dm), win_bf, cw,
                 conv_b.reshape(nl, 1, di), wdtr, wb, wc, wdt_bf,
                 dt_proj_b.reshape(nl, 1, di), alogT, D.reshape(nl, 1, di),
                 wo_bf, norm_f_w, seg=seg)
    logits = _lmhead(hf, embed, vtile=3200)
    return logits.reshape(bsz, seg, v)


# speedup vs baseline: 1.0227x; 1.0057x over previous
import functools

import jax
import jax.numpy as jnp
from jax.experimental import pallas as pl
from jax.experimental.pallas import tpu as pltpu

_INTERPRET = False

_LOG2E = 1.4426950408889634
_CONTRACT_LAST = (((1,), (1,)), ((), ()))
_VMEM_LIM = 110 * 1024 * 1024


def _bf(x):
    return x.astype(jnp.bfloat16)


def _silu(v):
    return v * jax.nn.sigmoid(v)


def _rms_bf16(xv, nw):
    ms = jnp.mean(xv * xv, axis=-1, keepdims=True)
    return _bf(xv * jax.lax.rsqrt(ms + 1e-5) * nw)


def _gather_body(ids_ref, emb_ref, win_ref, wo_ref, wdt_ref, xp_ref,
                 out_ref, winb_ref, wob_ref, wdtb_ref, xpb_ref, sem, *, nch):
    n = out_ref.shape[0]
    c = pl.program_id(0)

    @pl.when(c == 0)
    def _():
        def issue(i, _):
            idx = ids_ref[i]
            pltpu.make_async_copy(emb_ref.at[pl.ds(idx, 1), :],
                                  out_ref.at[pl.ds(i, 1), :], sem).start()
            return 0

        jax.lax.fori_loop(0, n, issue, 0)
        xpb_ref[...] = xp_ref[...].astype(jnp.bfloat16)

    winb_ref[...] = win_ref[...].astype(jnp.bfloat16)
    wob_ref[...] = wo_ref[...].astype(jnp.bfloat16)
    wdtb_ref[...] = wdt_ref[...].astype(jnp.bfloat16)

    @pl.when(c == nch - 1)
    def _():
        def drain(i, _):
            pltpu.make_async_copy(emb_ref.at[pl.ds(0, 1), :],
                                  out_ref.at[pl.ds(0, 1), :], sem).wait()
            return 0

        jax.lax.fori_loop(0, n, drain, 0)


def _embed_gather(ids_flat, embed, win_f, wo_f, wdt_f, xp_f, *, nch=8):
    m = ids_flat.shape[0]
    dm = embed.shape[1]
    rw, cw_ = win_f.shape
    ro, co = wo_f.shape
    rd, cd = wdt_f.shape
    kern = functools.partial(_gather_body, nch=nch)
    return pl.pallas_call(
        kern,
        grid=(nch,),
        in_specs=[pl.BlockSpec(memory_space=pltpu.SMEM),
                  pl.BlockSpec(memory_space=pl.ANY),
                  pl.BlockSpec((rw // nch, cw_), lambda c: (c, 0)),
                  pl.BlockSpec((ro // nch, co), lambda c: (c, 0)),
                  pl.BlockSpec((rd // nch, cd), lambda c: (c, 0)),
                  pl.BlockSpec(xp_f.shape, lambda c: (0, 0))],
        out_specs=[pl.BlockSpec((m, dm), lambda c: (0, 0)),
                   pl.BlockSpec((rw // nch, cw_), lambda c: (c, 0)),
                   pl.BlockSpec((ro // nch, co), lambda c: (c, 0)),
                   pl.BlockSpec((rd // nch, cd), lambda c: (c, 0)),
                   pl.BlockSpec(xp_f.shape, lambda c: (0, 0))],
        out_shape=[jax.ShapeDtypeStruct((m, dm), jnp.float32),
                   jax.ShapeDtypeStruct(win_f.shape, jnp.bfloat16),
                   jax.ShapeDtypeStruct(wo_f.shape, jnp.bfloat16),
                   jax.ShapeDtypeStruct(wdt_f.shape, jnp.bfloat16),
                   jax.ShapeDtypeStruct(xp_f.shape, jnp.bfloat16)],
        scratch_shapes=[pltpu.SemaphoreType.DMA],
        compiler_params=pltpu.CompilerParams(
            dimension_semantics=("arbitrary",),
            vmem_limit_bytes=_VMEM_LIM,
        ),
        name="embed_gather_prep",
        interpret=_INTERPRET,
    )(ids_flat, embed, win_f, wo_f, wdt_f, xp_f)


def _layers_body(x0_ref, nw_ref, win_ref, cw_ref, cb_ref, xp_ref,
                 wdt_ref, dtb_ref, alog_ref, d_ref, wo_ref, wf_ref,
                 hf_ref,
                 x_ref, u_ref, zs_ref, dts_ref, yg_ref,
                 bmt_ref, cmt_ref, bm8_ref, cm8_ref,
                 *, seg, nl):
    l = pl.program_id(0)
    mb = u_ref.shape[0]
    di = u_ref.shape[2]
    m = mb * 8
    ds = alog_ref.shape[1]
    nblk = seg // 8
    nbatch = m // seg

    @pl.when(l == 0)
    def _():
        x_ref[...] = x0_ref[...]

    hb = _rms_bf16(x_ref[...], nw_ref[0])
    xz_u = jax.lax.dot_general(hb, win_ref[0, 0:di], _CONTRACT_LAST,
                               preferred_element_type=jnp.float32)
    xz_z = jax.lax.dot_general(hb, win_ref[0, di:2 * di], _CONTRACT_LAST,
                               preferred_element_type=jnp.float32)
    dc = cw_ref.shape[1]
    row = jax.lax.broadcasted_iota(jnp.int32, (m, 1), 0)
    pos = jax.lax.rem(row, seg)
    uc = xz_u * cw_ref[0, dc - 1:dc, :]
    for s in range(1, dc):
        shifted = jnp.concatenate(
            [jnp.zeros((s, di), jnp.float32), xz_u[:-s, :]], axis=0)
        shifted = jnp.where(pos >= s, shifted, 0.0)
        uc = uc + shifted * cw_ref[0, dc - 1 - s:dc - s, :]
    u = _silu(uc + cb_ref[0])
    u_ref[...] = u.reshape(mb, 8, di)
    zs_ref[...] = _bf(_silu(xz_z)).reshape(mb, 8, di)

    ub = _bf(u)
    dtrk = xp_ref.shape[1] - 2 * ds
    dtr = jax.lax.dot_general(ub, xp_ref[0, 0:dtrk], _CONTRACT_LAST,
                              preferred_element_type=jnp.float32)
    bmt_ref[...] = jax.lax.dot_general(xp_ref[0, dtrk:dtrk + ds], ub,
                                       _CONTRACT_LAST,
                                       preferred_element_type=jnp.float32)
    cmt_ref[...] = jax.lax.dot_general(xp_ref[0, dtrk + ds:dtrk + 2 * ds], ub,
                                       _CONTRACT_LAST,
                                       preferred_element_type=jnp.float32)
    dtx = jax.lax.dot_general(_bf(dtr), wdt_ref[0], _CONTRACT_LAST,
                              preferred_element_type=jnp.float32)
    dtx = dtx + dtb_ref[0]
    dt = jnp.where(dtx > 20.0, dtx, jnp.log1p(jnp.exp(dtx)))
    dts_ref[...] = dt.reshape(mb, 8, di)
    for i in range(mb):
        bm8_ref[i] = bmt_ref[:, 8 * i:8 * i + 8]
        cm8_ref[i] = cmt_ref[:, 8 * i:8 * i + 8]

    a_sc = (-_LOG2E) * jnp.exp(alog_ref[0])
    dvec = d_ref[0]

    nrep = di // 128

    def batch_block(base, h):
        dt8 = dts_ref[base]
        u8 = u_ref[base]
        bc8 = bm8_ref[base]
        cc8 = cm8_ref[base]
        ys = []
        for j in range(8):
            dt_row = dt8[j:j + 1, :]
            dtb = pltpu.repeat(jnp.broadcast_to(dt_row, (8, di)), 2, axis=0)
            dtu = dt_row * u8[j:j + 1, :]
            dtub = pltpu.repeat(jnp.broadcast_to(dtu, (8, di)), 2, axis=0)
            bcf = pltpu.repeat(
                jnp.broadcast_to(bc8[:, j:j + 1], (ds, 128)), nrep, axis=1)
            ccf = pltpu.repeat(
                jnp.broadcast_to(cc8[:, j:j + 1], (ds, 128)), nrep, axis=1)
            a = jnp.exp2(a_sc * dtb)
            h = a * h + bcf * dtub
            ys.append(jnp.sum(ccf * h, axis=0, keepdims=True))
        y8 = jnp.concatenate(ys, axis=0)
        yg_ref[base] = _bf((y8 + u8 * dvec) *
                           zs_ref[base].astype(jnp.float32))
        return h

    def body(blk, carry):
        return tuple(
            batch_block(b * nblk + blk, carry[b]) for b in range(nbatch))

    z = jnp.zeros((ds, di), jnp.float32)
    jax.lax.fori_loop(0, nblk, body, (z,) * nbatch)

    xn = x_ref[...] + jax.lax.dot_general(
        yg_ref[...].reshape(m, di), wo_ref[0], _CONTRACT_LAST,
        preferred_element_type=jnp.float32)
    x_ref[...] = xn

    @pl.when(l == nl - 1)
    def _():
        hf_ref[...] = _rms_bf16(xn, wf_ref[0])


def _layers(x0, norm_w, win_bf, cw, cb, xp_bf, wdt_bf, dtb,
            alogT, dmat, wo_bf, norm_f_w, *, seg):
    m, dm = x0.shape
    nl, ds, di = alogT.shape
    dtrk = wdt_bf.shape[2]
    nxp = xp_bf.shape[1]
    dc = cw.shape[1]
    mb = m // 8
    kern = functools.partial(_layers_body, seg=seg, nl=nl)
    return pl.pallas_call(
        kern,
        grid=(nl,),
        in_specs=[
            pl.BlockSpec((m, dm), lambda l: (0, 0)),
            pl.BlockSpec((1, 1, dm), lambda l: (l, 0, 0)),
            pl.BlockSpec((1, 2 * di, dm), lambda l: (l, 0, 0)),
            pl.BlockSpec((1, dc, di), lambda l: (l, 0, 0)),
            pl.BlockSpec((1, 1, di), lambda l: (l, 0, 0)),
            pl.BlockSpec((1, nxp, di), lambda l: (l, 0, 0)),
            pl.BlockSpec((1, di, dtrk), lambda l: (l, 0, 0)),
            pl.BlockSpec((1, 1, di), lambda l: (l, 0, 0)),
            pl.BlockSpec((1, ds, di), lambda l: (l, 0, 0)),
            pl.BlockSpec((1, 1, di), lambda l: (l, 0, 0)),
            pl.BlockSpec((1, dm, di), lambda l: (l, 0, 0)),
            pl.BlockSpec((1, 1, dm), lambda l: (0, 0, 0)),
        ],
        out_specs=pl.BlockSpec((m, dm), lambda l: (0, 0)),
        out_shape=jax.ShapeDtypeStruct((m, dm), jnp.bfloat16),
        scratch_shapes=[
            pltpu.VMEM((m, dm), jnp.float32),
            pltpu.VMEM((mb, 8, di), jnp.float32),
            pltpu.VMEM((mb, 8, di), jnp.bfloat16),
            pltpu.VMEM((mb, 8, di), jnp.float32),
            pltpu.VMEM((mb, 8, di), jnp.bfloat16),
            pltpu.VMEM((ds, m), jnp.float32),
            pltpu.VMEM((ds, m), jnp.float32),
            pltpu.VMEM((mb, ds, 8), jnp.float32),
            pltpu.VMEM((mb, ds, 8), jnp.float32),
        ],
        compiler_params=pltpu.CompilerParams(
            dimension_semantics=("arbitrary",),
            vmem_limit_bytes=_VMEM_LIM,
        ),
        name="mamba_layers",
        interpret=_INTERPRET,
    )(x0, norm_w, win_bf, cw, cb, xp_bf, wdt_bf, dtb,
      alogT, dmat, wo_bf, norm_f_w.reshape(1, 1, dm))


def _lmhead_body(h_ref, e_ref, o_ref):
    o_ref[...] = jax.lax.dot_general(
        h_ref[...], _bf(e_ref[...]), _CONTRACT_LAST,
        preferred_element_type=jnp.float32)


def _lmhead(hf, embed, *, vtile):
    m, dm = hf.shape
    v = embed.shape[0]
    nv = v // vtile
    return pl.pallas_call(
        _lmhead_body,
        grid=(nv,),
        in_specs=[
            pl.BlockSpec((m, dm), lambda i: (0, 0)),
            pl.BlockSpec((vtile, dm), lambda i: (i, 0)),
        ],
        out_specs=pl.BlockSpec((m, vtile), lambda i: (0, i)),
        out_shape=jax.ShapeDtypeStruct((m, v), jnp.float32),
        compiler_params=pltpu.CompilerParams(
            dimension_semantics=("arbitrary",),
            vmem_limit_bytes=_VMEM_LIM,
        ),
        name="lm_head",
        interpret=_INTERPRET,
    )(hf, embed)


def kernel(input_ids, embed, norm_w, in_proj_w, conv_w, conv_b, x_proj_w,
           dt_proj_w, dt_proj_b, A_log, D, out_proj_w, norm_f_w):
    bsz, seg = input_ids.shape
    v, dm = embed.shape
    nl, di, ds = A_log.shape
    dtr = dt_proj_w.shape[2]
    m = bsz * seg

    cw = jnp.swapaxes(conv_w[:, :, 0, :], 1, 2)
    alogT = jnp.swapaxes(A_log, 1, 2)
    nxp = x_proj_w.shape[1]

    x0, win_bf2, wo_bf2, wdt_bf2, xp_bf2 = _embed_gather(
        input_ids.reshape(m), embed,
        in_proj_w.reshape(nl * 2 * di, dm), out_proj_w.reshape(nl * dm, di),
        dt_proj_w.reshape(nl * di, dtr), x_proj_w.reshape(nl * nxp, di))
    hf = _layers(x0, norm_w.reshape(nl, 1, dm),
                 win_bf2.reshape(nl, 2 * di, dm), cw,
                 conv_b.reshape(nl, 1, di), xp_bf2.reshape(nl, nxp, di),
                 wdt_bf2.reshape(nl, di, dtr),
                 dt_proj_b.reshape(nl, 1, di), alogT, D.reshape(nl, 1, di),
                 wo_bf2.reshape(nl, dm, di), norm_f_w, seg=seg)
    logits = _lmhead(hf, embed, vtile=3200)
    return logits.reshape(bsz, seg, v)
